# Initial kernel scaffold; baseline (speedup 1.0000x reference)
import jax
import jax.numpy as jnp
from jax.experimental import pallas as pl


def kernel(x):
    raise NotImplementedError("write your pallas kernel here")


# speedup vs baseline: 8.9712x; 8.9712x over previous
import functools

import jax
import jax.numpy as jnp
from jax import lax
from jax.experimental import pallas as pl
from jax.experimental.pallas import tpu as pltpu
from jax.experimental.pallas import tpu_sc as plsc

_B = 16
_N = 96 * 56 * 56
_K = int(0.05 * _N)
_HALF = _N // 2
_CHUNK = _HALF // 8
_NCHUNK = _HALF // _CHUNK
_NB = (2048, 2048, 1024)


def _make_sc_hist(pass_idx):
    nb = _NB[pass_idx - 1]
    mesh = plsc.VectorSubcoreMesh(core_axis_name="c", subcore_axis_name="s")

    @functools.partial(
        pl.kernel,
        mesh=mesh,
        compiler_params=pltpu.CompilerParams(needs_layout_passes=False),
        out_type=jax.ShapeDtypeStruct((_B, 2, 16 * nb), jnp.int32),
        scratch_types=[
            pltpu.VMEM((_CHUNK,), jnp.int32),
            pltpu.VMEM((_CHUNK,), jnp.int32),
            pltpu.VMEM((16 * nb,), jnp.int32),
            pltpu.VMEM((16,), jnp.int32),
            pltpu.SemaphoreType.DMA,
            pltpu.SemaphoreType.DMA,
        ],
    )
    def hist_kernel(x_hbm, pref_hbm, out_hbm, buf0, buf1, hist, pvec, sem0,
                    sem1):
        row = lax.axis_index("s")
        half = lax.axis_index("c")
        base = half * _HALF

        bufs = (buf0, buf1)
        sems = (sem0, sem1)
        cur = pltpu.async_copy(x_hbm.at[row, pl.ds(base, _CHUNK)], buf0, sem0)

        if pass_idx > 1:
            pltpu.sync_copy(pref_hbm, pvec)
            rowv = jnp.zeros((16,), jnp.int32) + row
            pbroad = plsc.load_gather(pvec, [rowv])

        zer = jnp.zeros((16,), jnp.int32)

        def zbody(i, carry):
            hist[pl.ds(i * 16, 16)] = zer
            return carry

        lax.fori_loop(0, nb, zbody, 0, unroll=8)

        lane_base = lax.iota(jnp.int32, 16) * nb
        ones = jnp.ones((16,), jnp.int32)

        for j in range(_NCHUNK):
            nxt = None
            if j + 1 < _NCHUNK:
                nxt = pltpu.async_copy(
                    x_hbm.at[row, pl.ds(base + (j + 1) * _CHUNK, _CHUNK)],
                    bufs[(j + 1) % 2], sems[(j + 1) % 2])
            cur.wait()
            buf = bufs[j % 2]

            def body(i, carry):
                b = buf[pl.ds(i * 16, 16)]
                key = b ^ (jnp.right_shift(b, 31) & jnp.int32(0x7FFFFFFF))
                if pass_idx == 1:
                    bin_ = jnp.right_shift(key, 21) + jnp.int32(1024)
                    plsc.addupdate_scatter(hist, [lane_base + bin_], ones)
                elif pass_idx == 2:
                    bin_ = jnp.right_shift(key, 10) & jnp.int32(0x7FF)
                    pm = jnp.right_shift(key, 21) == pbroad
                    plsc.addupdate_scatter(hist, [lane_base + bin_], ones,
                                           mask=pm)
                else:
                    bin_ = key & jnp.int32(0x3FF)
                    pm = jnp.right_shift(key, 10) == pbroad
                    plsc.addupdate_scatter(hist, [lane_base + bin_], ones,
                                           mask=pm)
                return carry

            lax.fori_loop(0, _CHUNK // 16, body, 0, unroll=4)
            cur = nxt

        pltpu.sync_copy(hist, out_hbm.at[row, half])

    return hist_kernel


def _loc_call(mode, hist4, rank, pref):
    nb = _NB[mode - 1]
    nbs = nb // 128

    def body(hist_ref, rank_ref, pref_ref, *outs):
        csum = hist_ref[0, 0].astype(jnp.float32)
        for i in range(1, 32):
            csum = csum + hist_ref[0, i].astype(jnp.float32)
        ls = csum
        d = 1
        while d < 128:
            ls = ls + jnp.concatenate(
                [ls[:, d:], jnp.zeros((nbs, d), jnp.float32)], axis=1)
            d *= 2
        tot = ls[:, 0:1]
        ts = tot
        d = 1
        while d < nbs:
            ts = ts + jnp.concatenate(
                [ts[d:, :], jnp.zeros((d, 1), jnp.float32)], axis=0)
            d *= 2
        s = ls + (ts - tot)

        def _sum11(a):
            return jnp.sum(jnp.sum(a, axis=1, keepdims=True), axis=0,
                           keepdims=True)

        r_in = rank_ref[0]
        bstar = _sum11((s >= r_in).astype(jnp.float32)) - 1.0
        bidx = (lax.broadcasted_iota(jnp.int32, (nbs, 128), 0) * 128
                + lax.broadcasted_iota(jnp.int32, (nbs, 128), 1)
                ).astype(jnp.float32)
        s_next = _sum11(jnp.where(bidx == bstar + 1.0, s, 0.0))
        b_i = bstar.astype(jnp.int32)
        if mode == 1:
            outs[0][0] = b_i - jnp.int32(1024)
            outs[1][0] = r_in - s_next
        elif mode == 2:
            outs[0][0] = pref_ref[0] * jnp.int32(2048) + b_i
            outs[1][0] = r_in - s_next
        else:
            key = pref_ref[0] * jnp.int32(1024) + b_i
            bits = key ^ (jnp.right_shift(key, 31) & jnp.int32(0x7FFFFFFF))
            outs[0][0] = lax.bitcast_convert_type(bits, jnp.float32)

    if mode < 3:
        out_shape = [
            jax.ShapeDtypeStruct((_B, 1, 1), jnp.int32),
            jax.ShapeDtypeStruct((_B, 1, 1), jnp.float32),
        ]
        out_specs = [
            pl.BlockSpec((1, 1, 1), lambda r: (r, 0, 0)),
            pl.BlockSpec((1, 1, 1), lambda r: (r, 0, 0)),
        ]
    else:
        out_shape = [jax.ShapeDtypeStruct((_B, 1, 1), jnp.float32)]
        out_specs = [pl.BlockSpec((1, 1, 1), lambda r: (r, 0, 0))]

    return pl.pallas_call(
        body,
        grid=(_B,),
        in_specs=[
            pl.BlockSpec((1, 32, nbs, 128), lambda r: (r, 0, 0, 0)),
            pl.BlockSpec((1, 1, 1), lambda r: (r, 0, 0)),
            pl.BlockSpec((1, 1, 1), lambda r: (r, 0, 0)),
        ],
        out_specs=out_specs,
        out_shape=out_shape,
    )(hist4, rank, pref)


def _mask_call(x3, thr):
    def body(x_ref, t_ref, o_ref):
        t = t_ref[...]
        xv = x_ref[...]
        o_ref[...] = xv * (xv >= t).astype(xv.dtype)

    nsub = _N // 128

    return pl.pallas_call(
        body,
        grid=(_B,),
        in_specs=[
            pl.BlockSpec((1, nsub, 128), lambda r: (r, 0, 0)),
            pl.BlockSpec((1, 1, 1), lambda r: (r, 0, 0)),
        ],
        out_specs=pl.BlockSpec((1, nsub, 128), lambda r: (r, 0, 0)),
        out_shape=jax.ShapeDtypeStruct((_B, nsub, 128), jnp.float32),
    )(x3, thr)


def kernel(x):
    assert x.shape == (_B, 96, 56, 56) and x.dtype == jnp.float32
    xi = lax.bitcast_convert_type(x, jnp.int32).reshape(_B, _N)
    z16 = jnp.zeros((_B,), jnp.int32)
    rank0 = jnp.full((_B, 1, 1), float(_K), jnp.float32)
    zp = jnp.zeros((_B, 1, 1), jnp.int32)

    h1 = _make_sc_hist(1)(xi, z16)
    p1, r1 = _loc_call(1, h1.reshape(_B, 32, _NB[0] // 128, 128), rank0, zp)
    h2 = _make_sc_hist(2)(xi, p1.reshape(_B))
    p2, r2 = _loc_call(2, h2.reshape(_B, 32, _NB[1] // 128, 128), r1, p1)
    h3 = _make_sc_hist(3)(xi, p2.reshape(_B))
    thr, = _loc_call(3, h3.reshape(_B, 32, _NB[2] // 128, 128), r2, p2)

    out3 = _mask_call(x.reshape(_B, _N // 128, 128), thr)
    return out3.reshape(x.shape)


# speedup vs baseline: 9.4834x; 1.0571x over previous
import functools

import jax
import jax.numpy as jnp
from jax import lax
from jax.experimental import pallas as pl
from jax.experimental.pallas import tpu as pltpu
from jax.experimental.pallas import tpu_sc as plsc

_B = 16
_N = 96 * 56 * 56
_K = int(0.05 * _N)
_HALF = _N // 2
_CHUNK = _HALF // 8
_NCHUNK = _HALF // _CHUNK
_NB = (2048, 2048, 1024)


def _make_sc_hist(pass_idx):
    nb = _NB[pass_idx - 1]
    mesh = plsc.VectorSubcoreMesh(core_axis_name="c", subcore_axis_name="s")

    @functools.partial(
        pl.kernel,
        mesh=mesh,
        compiler_params=pltpu.CompilerParams(needs_layout_passes=False),
        out_type=jax.ShapeDtypeStruct((_B, 2, nb), jnp.int32),
        scratch_types=[
            pltpu.VMEM((_CHUNK,), jnp.float32),
            pltpu.VMEM((_CHUNK,), jnp.float32),
            pltpu.VMEM((16 * nb,), jnp.int32),
            pltpu.VMEM((nb,), jnp.int32),
            pltpu.VMEM((16,), jnp.int32),
            pltpu.SemaphoreType.DMA,
            pltpu.SemaphoreType.DMA,
        ],
    )
    def hist_kernel(x_hbm, pref_hbm, out_hbm, buf0, buf1, hist, red, pvec,
                    sem0, sem1):
        row = lax.axis_index("s")
        half = lax.axis_index("c")
        base = half * _HALF

        bufs = (buf0, buf1)
        sems = (sem0, sem1)
        cur = pltpu.async_copy(x_hbm.at[row, pl.ds(base, _CHUNK)], buf0, sem0)

        if pass_idx > 1:
            pltpu.sync_copy(pref_hbm, pvec)
            rowv = jnp.zeros((16,), jnp.int32) + row
            pbroad = plsc.load_gather(pvec, [rowv])

        zer = jnp.zeros((16,), jnp.int32)

        def zbody(i, carry):
            hist[pl.ds(i * 16, 16)] = zer
            return carry

        lax.fori_loop(0, nb, zbody, 0, unroll=8)

        lane_iota = lax.iota(jnp.int32, 16)
        ones = jnp.ones((16,), jnp.int32)

        for j in range(_NCHUNK):
            nxt = None
            if j + 1 < _NCHUNK:
                nxt = pltpu.async_copy(
                    x_hbm.at[row, pl.ds(base + (j + 1) * _CHUNK, _CHUNK)],
                    bufs[(j + 1) % 2], sems[(j + 1) % 2])
            cur.wait()
            buf = bufs[j % 2]

            def body(i, carry):
                v = buf[pl.ds(i * 16, 16)]
                b = plsc.bitcast(v, jnp.int32)
                key = b ^ (jnp.right_shift(b, 31) & jnp.int32(0x7FFFFFFF))
                if pass_idx == 1:
                    bin_ = jnp.right_shift(key, 21) + jnp.int32(1024)
                    plsc.addupdate_scatter(hist, [bin_ * 16 + lane_iota], ones)
                elif pass_idx == 2:
                    bin_ = jnp.right_shift(key, 10) & jnp.int32(0x7FF)
                    pm = jnp.right_shift(key, 21) == pbroad
                    plsc.addupdate_scatter(hist, [bin_ * 16 + lane_iota], ones,
                                           mask=pm)
                else:
                    bin_ = key & jnp.int32(0x3FF)
                    pm = jnp.right_shift(key, 10) == pbroad
                    plsc.addupdate_scatter(hist, [bin_ * 16 + lane_iota], ones,
                                           mask=pm)
                return carry

            lax.fori_loop(0, _CHUNK // 16, body, 0, unroll=4)
            cur = nxt

        gidx = lane_iota * 16

        def rbody(g, carry):
            acc = jnp.zeros((16,), jnp.int32)
            for l in range(16):
                acc = acc + plsc.load_gather(hist, [g * 256 + gidx + l])
            red[pl.ds(g * 16, 16)] = acc
            return carry

        lax.fori_loop(0, nb // 16, rbody, 0, unroll=2)
        pltpu.sync_copy(red, out_hbm.at[row, half])

    return hist_kernel


def _loc_call(mode, hist4, rank, pref):
    nb = _NB[mode - 1]
    nbs = nb // 128

    def body(hist_ref, rank_ref, pref_ref, *outs):
        csum = (hist_ref[0, 0] + hist_ref[0, 1]).astype(jnp.float32)
        ls = csum
        d = 1
        while d < 128:
            ls = ls + jnp.concatenate(
                [ls[:, d:], jnp.zeros((nbs, d), jnp.float32)], axis=1)
            d *= 2
        tot = ls[:, 0:1]
        ts = tot
        d = 1
        while d < nbs:
            ts = ts + jnp.concatenate(
                [ts[d:, :], jnp.zeros((d, 1), jnp.float32)], axis=0)
            d *= 2
        s = ls + (ts - tot)

        def _sum11(a):
            return jnp.sum(jnp.sum(a, axis=1, keepdims=True), axis=0,
                           keepdims=True)

        r_in = rank_ref[0]
        bstar = _sum11((s >= r_in).astype(jnp.float32)) - 1.0
        bidx = (lax.broadcasted_iota(jnp.int32, (nbs, 128), 0) * 128
                + lax.broadcasted_iota(jnp.int32, (nbs, 128), 1)
                ).astype(jnp.float32)
        s_next = _sum11(jnp.where(bidx == bstar + 1.0, s, 0.0))
        b_i = bstar.astype(jnp.int32)
        if mode == 1:
            outs[0][0] = b_i - jnp.int32(1024)
            outs[1][0] = r_in - s_next
        elif mode == 2:
            outs[0][0] = pref_ref[0] * jnp.int32(2048) + b_i
            outs[1][0] = r_in - s_next
        else:
            key = pref_ref[0] * jnp.int32(1024) + b_i
            bits = key ^ (jnp.right_shift(key, 31) & jnp.int32(0x7FFFFFFF))
            outs[0][0] = lax.bitcast_convert_type(bits, jnp.float32)

    if mode < 3:
        out_shape = [
            jax.ShapeDtypeStruct((_B, 1, 1), jnp.int32),
            jax.ShapeDtypeStruct((_B, 1, 1), jnp.float32),
        ]
        out_specs = [
            pl.BlockSpec((1, 1, 1), lambda r: (r, 0, 0)),
            pl.BlockSpec((1, 1, 1), lambda r: (r, 0, 0)),
        ]
    else:
        out_shape = [jax.ShapeDtypeStruct((_B, 1, 1), jnp.float32)]
        out_specs = [pl.BlockSpec((1, 1, 1), lambda r: (r, 0, 0))]

    return pl.pallas_call(
        body,
        grid=(_B,),
        in_specs=[
            pl.BlockSpec((1, 2, nbs, 128), lambda r: (r, 0, 0, 0)),
            pl.BlockSpec((1, 1, 1), lambda r: (r, 0, 0)),
            pl.BlockSpec((1, 1, 1), lambda r: (r, 0, 0)),
        ],
        out_specs=out_specs,
        out_shape=out_shape,
    )(hist4, rank, pref)


def _mask_call(x3, thr):
    def body(x_ref, t_ref, o_ref):
        t = t_ref[...]
        xv = x_ref[...]
        o_ref[...] = xv * (xv >= t).astype(xv.dtype)

    nsub = _N // 128

    return pl.pallas_call(
        body,
        grid=(_B,),
        in_specs=[
            pl.BlockSpec((1, nsub, 128), lambda r: (r, 0, 0)),
            pl.BlockSpec((1, 1, 1), lambda r: (r, 0, 0)),
        ],
        out_specs=pl.BlockSpec((1, nsub, 128), lambda r: (r, 0, 0)),
        out_shape=jax.ShapeDtypeStruct((_B, nsub, 128), jnp.float32),
    )(x3, thr)


def kernel(x):
    assert x.shape == (_B, 96, 56, 56) and x.dtype == jnp.float32
    xr = x.reshape(_B, _N)
    z16 = jnp.zeros((_B,), jnp.int32)
    rank0 = jnp.full((_B, 1, 1), float(_K), jnp.float32)
    zp = jnp.zeros((_B, 1, 1), jnp.int32)

    h1 = _make_sc_hist(1)(xr, z16)
    p1, r1 = _loc_call(1, h1.reshape(_B, 2, _NB[0] // 128, 128), rank0, zp)
    h2 = _make_sc_hist(2)(xr, p1.reshape(_B))
    p2, r2 = _loc_call(2, h2.reshape(_B, 2, _NB[1] // 128, 128), r1, p1)
    h3 = _make_sc_hist(3)(xr, p2.reshape(_B))
    thr, = _loc_call(3, h3.reshape(_B, 2, _NB[2] // 128, 128), r2, p2)

    out3 = _mask_call(x.reshape(_B, _N // 128, 128), thr)
    return out3.reshape(x.shape)


# speedup vs baseline: 9.6917x; 1.0220x over previous
import functools

import jax
import jax.numpy as jnp
from jax import lax
from jax.experimental import pallas as pl
from jax.experimental.pallas import tpu as pltpu
from jax.experimental.pallas import tpu_sc as plsc

_B = 16
_N = 96 * 56 * 56
_K = int(0.05 * _N)
_HALF = _N // 2
_CHUNK = _HALF // 8
_NCHUNK = _HALF // _CHUNK
_NB = (2048, 2048, 1024)


def _make_sc_hist(pass_idx):
    nb = _NB[pass_idx - 1]
    mesh = plsc.VectorSubcoreMesh(core_axis_name="c", subcore_axis_name="s")

    @functools.partial(
        pl.kernel,
        mesh=mesh,
        compiler_params=pltpu.CompilerParams(needs_layout_passes=False),
        out_type=jax.ShapeDtypeStruct((_B, 2, nb), jnp.int32),
        scratch_types=[
            pltpu.VMEM((_CHUNK,), jnp.float32),
            pltpu.VMEM((_CHUNK,), jnp.float32),
            pltpu.VMEM((nb,), jnp.int32),
            pltpu.VMEM((nb,), jnp.int32),
            pltpu.VMEM((nb,), jnp.int32),
            pltpu.VMEM((nb,), jnp.int32),
            pltpu.VMEM((nb,), jnp.int32),
            pltpu.VMEM((16,), jnp.int32),
            pltpu.SemaphoreType.DMA,
            pltpu.SemaphoreType.DMA,
        ],
    )
    def hist_kernel(x_hbm, pref_hbm, out_hbm, buf0, buf1, h0, h1, h2, h3,
                    red, pvec, sem0, sem1):
        row = lax.axis_index("s")
        half = lax.axis_index("c")
        base = half * _HALF

        bufs = (buf0, buf1)
        sems = (sem0, sem1)
        cur = pltpu.async_copy(x_hbm.at[row, pl.ds(base, _CHUNK)], buf0, sem0)

        if pass_idx > 1:
            pltpu.sync_copy(pref_hbm, pvec)
            rowv = jnp.zeros((16,), jnp.int32) + row
            pbroad = plsc.load_gather(pvec, [rowv])

        hists = (h0, h1, h2, h3)
        zer = jnp.zeros((16,), jnp.int32)

        def zbody(i, carry):
            for h in hists:
                h[pl.ds(i * 16, 16)] = zer
            return carry

        lax.fori_loop(0, nb // 16, zbody, 0, unroll=4)

        ones = jnp.ones((16,), jnp.int32)

        for j in range(_NCHUNK):
            nxt = None
            if j + 1 < _NCHUNK:
                nxt = pltpu.async_copy(
                    x_hbm.at[row, pl.ds(base + (j + 1) * _CHUNK, _CHUNK)],
                    bufs[(j + 1) % 2], sems[(j + 1) % 2])
            cur.wait()
            buf = bufs[j % 2]

            def body(i, carry):
                for u in range(4):
                    v = buf[pl.ds(i * 64 + u * 16, 16)]
                    b = plsc.bitcast(v, jnp.int32)
                    key = b ^ (jnp.right_shift(b, 31) & jnp.int32(0x7FFFFFFF))
                    if pass_idx == 1:
                        bin_ = jnp.right_shift(key, 21) + jnp.int32(1024)
                        plsc.addupdate_scatter(hists[u], [bin_], ones)
                    elif pass_idx == 2:
                        bin_ = jnp.right_shift(key, 10) & jnp.int32(0x7FF)
                        pm = jnp.right_shift(key, 21) == pbroad
                        plsc.addupdate_scatter(hists[u], [bin_], ones, mask=pm)
                    else:
                        bin_ = key & jnp.int32(0x3FF)
                        pm = jnp.right_shift(key, 10) == pbroad
                        plsc.addupdate_scatter(hists[u], [bin_], ones, mask=pm)
                return carry

            lax.fori_loop(0, _CHUNK // 64, body, 0, unroll=2)
            cur = nxt

        def rbody(g, carry):
            sl = pl.ds(g * 16, 16)
            red[sl] = ((h0[sl] + h1[sl]) + (h2[sl] + h3[sl]))
            return carry

        lax.fori_loop(0, nb // 16, rbody, 0, unroll=4)
        pltpu.sync_copy(red, out_hbm.at[row, half])

    return hist_kernel


def _loc_call(mode, hist4, rank, pref):
    nb = _NB[mode - 1]
    nbs = nb // 128

    def body(hist_ref, rank_ref, pref_ref, *outs):
        csum = (hist_ref[0, 0] + hist_ref[0, 1]).astype(jnp.float32)
        ls = csum
        d = 1
        while d < 128:
            ls = ls + jnp.concatenate(
                [ls[:, d:], jnp.zeros((nbs, d), jnp.float32)], axis=1)
            d *= 2
        tot = ls[:, 0:1]
        ts = tot
        d = 1
        while d < nbs:
            ts = ts + jnp.concatenate(
                [ts[d:, :], jnp.zeros((d, 1), jnp.float32)], axis=0)
            d *= 2
        s = ls + (ts - tot)

        def _sum11(a):
            return jnp.sum(jnp.sum(a, axis=1, keepdims=True), axis=0,
                           keepdims=True)

        r_in = rank_ref[0]
        bstar = _sum11((s >= r_in).astype(jnp.float32)) - 1.0
        bidx = (lax.broadcasted_iota(jnp.int32, (nbs, 128), 0) * 128
                + lax.broadcasted_iota(jnp.int32, (nbs, 128), 1)
                ).astype(jnp.float32)
        s_next = _sum11(jnp.where(bidx == bstar + 1.0, s, 0.0))
        b_i = bstar.astype(jnp.int32)
        if mode == 1:
            outs[0][0] = b_i - jnp.int32(1024)
            outs[1][0] = r_in - s_next
        elif mode == 2:
            outs[0][0] = pref_ref[0] * jnp.int32(2048) + b_i
            outs[1][0] = r_in - s_next
        else:
            key = pref_ref[0] * jnp.int32(1024) + b_i
            bits = key ^ (jnp.right_shift(key, 31) & jnp.int32(0x7FFFFFFF))
            outs[0][0] = lax.bitcast_convert_type(bits, jnp.float32)

    if mode < 3:
        out_shape = [
            jax.ShapeDtypeStruct((_B, 1, 1), jnp.int32),
            jax.ShapeDtypeStruct((_B, 1, 1), jnp.float32),
        ]
        out_specs = [
            pl.BlockSpec((1, 1, 1), lambda r: (r, 0, 0)),
            pl.BlockSpec((1, 1, 1), lambda r: (r, 0, 0)),
        ]
    else:
        out_shape = [jax.ShapeDtypeStruct((_B, 1, 1), jnp.float32)]
        out_specs = [pl.BlockSpec((1, 1, 1), lambda r: (r, 0, 0))]

    return pl.pallas_call(
        body,
        grid=(_B,),
        in_specs=[
            pl.BlockSpec((1, 2, nbs, 128), lambda r: (r, 0, 0, 0)),
            pl.BlockSpec((1, 1, 1), lambda r: (r, 0, 0)),
            pl.BlockSpec((1, 1, 1), lambda r: (r, 0, 0)),
        ],
        out_specs=out_specs,
        out_shape=out_shape,
    )(hist4, rank, pref)


def _mask_call(x3, thr):
    def body(x_ref, t_ref, o_ref):
        t = t_ref[...]
        xv = x_ref[...]
        o_ref[...] = xv * (xv >= t).astype(xv.dtype)

    nsub = _N // 128

    return pl.pallas_call(
        body,
        grid=(_B,),
        in_specs=[
            pl.BlockSpec((1, nsub, 128), lambda r: (r, 0, 0)),
            pl.BlockSpec((1, 1, 1), lambda r: (r, 0, 0)),
        ],
        out_specs=pl.BlockSpec((1, nsub, 128), lambda r: (r, 0, 0)),
        out_shape=jax.ShapeDtypeStruct((_B, nsub, 128), jnp.float32),
    )(x3, thr)


def kernel(x):
    assert x.shape == (_B, 96, 56, 56) and x.dtype == jnp.float32
    xr = x.reshape(_B, _N)
    z16 = jnp.zeros((_B,), jnp.int32)
    rank0 = jnp.full((_B, 1, 1), float(_K), jnp.float32)
    zp = jnp.zeros((_B, 1, 1), jnp.int32)

    h1 = _make_sc_hist(1)(xr, z16)
    p1, r1 = _loc_call(1, h1.reshape(_B, 2, _NB[0] // 128, 128), rank0, zp)
    h2 = _make_sc_hist(2)(xr, p1.reshape(_B))
    p2, r2 = _loc_call(2, h2.reshape(_B, 2, _NB[1] // 128, 128), r1, p1)
    h3 = _make_sc_hist(3)(xr, p2.reshape(_B))
    thr, = _loc_call(3, h3.reshape(_B, 2, _NB[2] // 128, 128), r2, p2)

    out3 = _mask_call(x.reshape(_B, _N // 128, 128), thr)
    return out3.reshape(x.shape)


# speedup vs baseline: 12.2271x; 1.2616x over previous
import functools

import jax
import jax.numpy as jnp
from jax import lax
from jax.experimental import pallas as pl
from jax.experimental.pallas import tpu as pltpu
from jax.experimental.pallas import tpu_sc as plsc

_B = 16
_N = 96 * 56 * 56
_K = int(0.05 * _N)
_HALF = _N // 2
_CHUNK = _HALF // 8
_NCHUNK = _HALF // _CHUNK
_NBINS = 65536


def _make_sc_hist(pass_idx):
    mesh = plsc.VectorSubcoreMesh(core_axis_name="c", subcore_axis_name="s")

    @functools.partial(
        pl.kernel,
        mesh=mesh,
        compiler_params=pltpu.CompilerParams(needs_layout_passes=False),
        out_type=jax.ShapeDtypeStruct((_B, 2, _NBINS), jnp.int32),
        scratch_types=[
            pltpu.VMEM((_CHUNK,), jnp.float32),
            pltpu.VMEM((_CHUNK,), jnp.float32),
            pltpu.VMEM((_NBINS,), jnp.int32),
            pltpu.VMEM((16,), jnp.int32),
            pltpu.SemaphoreType.DMA,
            pltpu.SemaphoreType.DMA,
        ],
    )
    def hist_kernel(x_hbm, pref_hbm, out_hbm, buf0, buf1, hist, pvec, sem0,
                    sem1):
        row = lax.axis_index("s")
        half = lax.axis_index("c")
        base = half * _HALF

        bufs = (buf0, buf1)
        sems = (sem0, sem1)
        cur = pltpu.async_copy(x_hbm.at[row, pl.ds(base, _CHUNK)], buf0, sem0)

        if pass_idx > 1:
            pltpu.sync_copy(pref_hbm, pvec)
            rowv = jnp.zeros((16,), jnp.int32) + row
            pbroad = plsc.load_gather(pvec, [rowv])

        zer = jnp.zeros((16,), jnp.int32)

        def zbody(i, carry):
            hist[pl.ds(i * 16, 16)] = zer
            return carry

        lax.fori_loop(0, _NBINS // 16, zbody, 0, unroll=8)

        ones = jnp.ones((16,), jnp.int32)

        for j in range(_NCHUNK):
            nxt = None
            if j + 1 < _NCHUNK:
                nxt = pltpu.async_copy(
                    x_hbm.at[row, pl.ds(base + (j + 1) * _CHUNK, _CHUNK)],
                    bufs[(j + 1) % 2], sems[(j + 1) % 2])
            cur.wait()
            buf = bufs[j % 2]

            if pass_idx == 1:

                def body(i, carry):
                    v = buf[pl.ds(i * 16, 16)]
                    b = plsc.bitcast(v, jnp.int32)
                    key = b ^ (jnp.right_shift(b, 31) & jnp.int32(0x7FFFFFFF))
                    bin_ = jnp.right_shift(key, 16) + jnp.int32(32768)
                    plsc.addupdate_scatter(hist, [bin_], ones)
                    return carry

                lax.fori_loop(0, _CHUNK // 16, body, 0, unroll=4)
            else:

                def body(i, carry):
                    bins = []
                    masks = []
                    anym = None
                    for u in range(4):
                        v = buf[pl.ds(i * 64 + u * 16, 16)]
                        b = plsc.bitcast(v, jnp.int32)
                        key = b ^ (jnp.right_shift(b, 31)
                                   & jnp.int32(0x7FFFFFFF))
                        pm = jnp.right_shift(key, 16) == pbroad
                        bins.append(key & jnp.int32(0xFFFF))
                        masks.append(pm)
                        anym = pm if anym is None else (anym | pm)
                    cnt = jnp.sum(anym.astype(jnp.int32))

                    def do_scatter(_):
                        for u in range(4):
                            plsc.addupdate_scatter(hist, [bins[u]], ones,
                                                   mask=masks[u])
                        return 0

                    lax.cond(cnt > 0, do_scatter, lambda _: 0, 0)
                    return carry

                lax.fori_loop(0, _CHUNK // 64, body, 0, unroll=2)
            cur = nxt

        pltpu.sync_copy(hist, out_hbm.at[row, half])

    return hist_kernel


def _loc_call(mode, hist4, rank, pref):
    nbs = _NBINS // 128

    def body(hist_ref, rank_ref, pref_ref, *outs):
        csum = (hist_ref[0, 0] + hist_ref[0, 1]).astype(jnp.float32)
        ls = csum
        d = 1
        while d < 128:
            ls = ls + jnp.concatenate(
                [ls[:, d:], jnp.zeros((nbs, d), jnp.float32)], axis=1)
            d *= 2
        tot = ls[:, 0:1]
        ts = tot
        d = 1
        while d < nbs:
            ts = ts + jnp.concatenate(
                [ts[d:, :], jnp.zeros((d, 1), jnp.float32)], axis=0)
            d *= 2
        s = ls + (ts - tot)

        def _sum11(a):
            return jnp.sum(jnp.sum(a, axis=1, keepdims=True), axis=0,
                           keepdims=True)

        r_in = rank_ref[0]
        bstar = _sum11((s >= r_in).astype(jnp.float32)) - 1.0
        bidx = (lax.broadcasted_iota(jnp.int32, (nbs, 128), 0) * 128
                + lax.broadcasted_iota(jnp.int32, (nbs, 128), 1)
                ).astype(jnp.float32)
        s_next = _sum11(jnp.where(bidx == bstar + 1.0, s, 0.0))
        b_i = bstar.astype(jnp.int32)
        if mode == 1:
            outs[0][0] = b_i - jnp.int32(32768)
            outs[1][0] = r_in - s_next
        else:
            key = pref_ref[0] * jnp.int32(65536) + b_i
            bits = key ^ (jnp.right_shift(key, 31) & jnp.int32(0x7FFFFFFF))
            outs[0][0] = lax.bitcast_convert_type(bits, jnp.float32)

    if mode == 1:
        out_shape = [
            jax.ShapeDtypeStruct((_B, 1, 1), jnp.int32),
            jax.ShapeDtypeStruct((_B, 1, 1), jnp.float32),
        ]
        out_specs = [
            pl.BlockSpec((1, 1, 1), lambda r: (r, 0, 0)),
            pl.BlockSpec((1, 1, 1), lambda r: (r, 0, 0)),
        ]
    else:
        out_shape = [jax.ShapeDtypeStruct((_B, 1, 1), jnp.float32)]
        out_specs = [pl.BlockSpec((1, 1, 1), lambda r: (r, 0, 0))]

    return pl.pallas_call(
        body,
        grid=(_B,),
        in_specs=[
            pl.BlockSpec((1, 2, nbs, 128), lambda r: (r, 0, 0, 0)),
            pl.BlockSpec((1, 1, 1), lambda r: (r, 0, 0)),
            pl.BlockSpec((1, 1, 1), lambda r: (r, 0, 0)),
        ],
        out_specs=out_specs,
        out_shape=out_shape,
    )(hist4, rank, pref)


def _mask_call(x, thr):
    def body(x_ref, t_ref, o_ref):
        t = t_ref[...]
        xv = x_ref[...]
        o_ref[...] = xv * (xv >= t).astype(xv.dtype)

    return pl.pallas_call(
        body,
        grid=(_B,),
        in_specs=[
            pl.BlockSpec((1, 96, 56, 56), lambda r: (r, 0, 0, 0)),
            pl.BlockSpec((1, 1, 1, 1), lambda r: (r, 0, 0, 0)),
        ],
        out_specs=pl.BlockSpec((1, 96, 56, 56), lambda r: (r, 0, 0, 0)),
        out_shape=jax.ShapeDtypeStruct((_B, 96, 56, 56), jnp.float32),
    )(x, thr)


def kernel(x):
    assert x.shape == (_B, 96, 56, 56) and x.dtype == jnp.float32
    xr = x.reshape(_B, _N)
    z16 = jnp.zeros((_B,), jnp.int32)
    rank0 = jnp.full((_B, 1, 1), float(_K), jnp.float32)
    zp = jnp.zeros((_B, 1, 1), jnp.int32)

    h1 = _make_sc_hist(1)(xr, z16)
    p1, r1 = _loc_call(1, h1.reshape(_B, 2, _NBINS // 128, 128), rank0, zp)
    h2 = _make_sc_hist(2)(xr, p1.reshape(_B))
    thr, = _loc_call(2, h2.reshape(_B, 2, _NBINS // 128, 128), r1, p1)

    return _mask_call(x, thr.reshape(_B, 1, 1, 1))


# speedup vs baseline: 13.3483x; 1.0917x over previous
import functools

import jax
import jax.numpy as jnp
from jax import lax
from jax.experimental import pallas as pl
from jax.experimental.pallas import tpu as pltpu
from jax.experimental.pallas import tpu_sc as plsc

_B = 16
_N = 96 * 56 * 56
_K = int(0.05 * _N)
_HALF = _N // 2
_CHUNK = _HALF // 8
_NCHUNK = _HALF // _CHUNK
_NBINS = 65536
_NBS = _NBINS // 128


def _make_sc_hist(pass_idx):
    mesh = plsc.VectorSubcoreMesh(core_axis_name="c", subcore_axis_name="s")

    @functools.partial(
        pl.kernel,
        mesh=mesh,
        compiler_params=pltpu.CompilerParams(needs_layout_passes=False),
        out_type=jax.ShapeDtypeStruct((_B, 2, _NBINS), jnp.int32),
        scratch_types=[
            pltpu.VMEM((_CHUNK,), jnp.float32),
            pltpu.VMEM((_CHUNK,), jnp.float32),
            pltpu.VMEM((_NBINS,), jnp.int32),
            pltpu.VMEM((16,), jnp.int32),
            pltpu.SemaphoreType.DMA,
            pltpu.SemaphoreType.DMA,
        ],
    )
    def hist_kernel(x_hbm, pref_hbm, out_hbm, buf0, buf1, hist, pvec, sem0,
                    sem1):
        row = lax.axis_index("s")
        half = lax.axis_index("c")
        base = half * _HALF

        bufs = (buf0, buf1)
        sems = (sem0, sem1)
        cur = pltpu.async_copy(x_hbm.at[row, pl.ds(base, _CHUNK)], buf0, sem0)

        if pass_idx > 1:
            pltpu.sync_copy(pref_hbm, pvec)
            rowv = jnp.zeros((16,), jnp.int32) + row
            pbroad = plsc.load_gather(pvec, [rowv])

        zer = jnp.zeros((16,), jnp.int32)

        def zbody(i, carry):
            hist[pl.ds(i * 16, 16)] = zer
            return carry

        lax.fori_loop(0, _NBINS // 16, zbody, 0, unroll=8)

        ones = jnp.ones((16,), jnp.int32)

        for j in range(_NCHUNK):
            nxt = None
            if j + 1 < _NCHUNK:
                nxt = pltpu.async_copy(
                    x_hbm.at[row, pl.ds(base + (j + 1) * _CHUNK, _CHUNK)],
                    bufs[(j + 1) % 2], sems[(j + 1) % 2])
            cur.wait()
            buf = bufs[j % 2]

            if pass_idx == 1:

                def body(i, carry):
                    v = buf[pl.ds(i * 16, 16)]
                    b = plsc.bitcast(v, jnp.int32)
                    key = b ^ (jnp.right_shift(b, 31) & jnp.int32(0x7FFFFFFF))
                    bin_ = jnp.right_shift(key, 16) + jnp.int32(32768)
                    plsc.addupdate_scatter(hist, [bin_], ones)
                    return carry

                lax.fori_loop(0, _CHUNK // 16, body, 0, unroll=4)
            else:

                def body(i, carry):
                    bins = []
                    masks = []
                    anym = None
                    for u in range(8):
                        v = buf[pl.ds(i * 128 + u * 16, 16)]
                        b = plsc.bitcast(v, jnp.int32)
                        key = b ^ (jnp.right_shift(b, 31)
                                   & jnp.int32(0x7FFFFFFF))
                        pm = jnp.right_shift(key, 16) == pbroad
                        bins.append(key & jnp.int32(0xFFFF))
                        masks.append(pm)
                        anym = pm if anym is None else (anym | pm)
                    cnt = jnp.sum(anym.astype(jnp.int32))

                    def do_scatter(_):
                        for u in range(8):
                            plsc.addupdate_scatter(hist, [bins[u]], ones,
                                                   mask=masks[u])
                        return 0

                    lax.cond(cnt > 0, do_scatter, lambda _: 0, 0)
                    return carry

                lax.fori_loop(0, _CHUNK // 128, body, 0, unroll=2)
            cur = nxt

        pltpu.sync_copy(hist, out_hbm.at[row, half])

    return hist_kernel


def _locate(hist_ref, r_in):
    c = (hist_ref[0, 0] + hist_ref[0, 1]).astype(jnp.float32)

    def _s11(a):
        return jnp.sum(jnp.sum(a, axis=1, keepdims=True), axis=0,
                       keepdims=True)

    rsum = jnp.sum(c, axis=1, keepdims=True)
    ts = rsum
    d = 1
    while d < _NBS:
        ts = ts + jnp.concatenate(
            [ts[d:, :], jnp.zeros((d, 1), jnp.float32)], axis=0)
        d *= 2
    i_star = jnp.sum(jnp.sum((ts >= r_in).astype(jnp.float32), axis=0,
                             keepdims=True), axis=1, keepdims=True) - 1.0
    ii = lax.broadcasted_iota(jnp.int32, (_NBS, 1), 0).astype(jnp.float32)
    ts_next = jnp.sum(jnp.where(ii == i_star + 1.0, ts, 0.0), axis=0,
                      keepdims=True)
    r2 = r_in - ts_next

    ii2 = lax.broadcasted_iota(jnp.int32, (_NBS, 128), 0).astype(jnp.float32)
    crow = jnp.sum(jnp.where(ii2 == i_star, c, 0.0), axis=0,
                   keepdims=True)
    ls = crow
    d = 1
    while d < 128:
        ls = ls + jnp.concatenate(
            [ls[:, d:], jnp.zeros((1, d), jnp.float32)], axis=1)
        d *= 2
    j_star = jnp.sum((ls >= r2).astype(jnp.float32), axis=1,
                     keepdims=True) - 1.0
    jj = lax.broadcasted_iota(jnp.int32, (1, 128), 1).astype(jnp.float32)
    ls_next = jnp.sum(jnp.where(jj == j_star + 1.0, ls, 0.0), axis=1,
                      keepdims=True)
    b_star = i_star * 128.0 + j_star
    above = ts_next + ls_next
    return b_star, above


def _loc1_call(hist4, rank):
    def body(hist_ref, rank_ref, p_out, r_out):
        r_in = rank_ref[0]
        b_star, above = _locate(hist_ref, r_in)
        p_out[0] = b_star.astype(jnp.int32) - jnp.int32(32768)
        r_out[0] = r_in - above

    return pl.pallas_call(
        body,
        grid=(_B,),
        in_specs=[
            pl.BlockSpec((1, 2, _NBS, 128), lambda r: (r, 0, 0, 0)),
            pl.BlockSpec((1, 1, 1), lambda r: (r, 0, 0)),
        ],
        out_specs=[
            pl.BlockSpec((1, 1, 1), lambda r: (r, 0, 0)),
            pl.BlockSpec((1, 1, 1), lambda r: (r, 0, 0)),
        ],
        out_shape=[
            jax.ShapeDtypeStruct((_B, 1, 1), jnp.int32),
            jax.ShapeDtypeStruct((_B, 1, 1), jnp.float32),
        ],
    )(hist4, rank)


def _loc2_mask_call(hist4, rank, pref, x):
    def body(hist_ref, rank_ref, pref_ref, x_ref, o_ref):
        r_in = rank_ref[0]
        b_star, _ = _locate(hist_ref, r_in)
        key = pref_ref[0] * jnp.int32(65536) + b_star.astype(jnp.int32)
        bits = key ^ (jnp.right_shift(key, 31) & jnp.int32(0x7FFFFFFF))
        thr = lax.bitcast_convert_type(bits, jnp.float32)
        t4 = jnp.reshape(thr, (1, 1, 1, 1))
        xv = x_ref[...]
        o_ref[...] = xv * (xv >= t4).astype(xv.dtype)

    return pl.pallas_call(
        body,
        grid=(_B,),
        in_specs=[
            pl.BlockSpec((1, 2, _NBS, 128), lambda r: (r, 0, 0, 0)),
            pl.BlockSpec((1, 1, 1), lambda r: (r, 0, 0)),
            pl.BlockSpec((1, 1, 1), lambda r: (r, 0, 0)),
            pl.BlockSpec((1, 96, 56, 56), lambda r: (r, 0, 0, 0)),
        ],
        out_specs=pl.BlockSpec((1, 96, 56, 56), lambda r: (r, 0, 0, 0)),
        out_shape=jax.ShapeDtypeStruct((_B, 96, 56, 56), jnp.float32),
    )(hist4, rank, pref, x)


def kernel(x):
    assert x.shape == (_B, 96, 56, 56) and x.dtype == jnp.float32
    xr = x.reshape(_B, _N)
    z16 = jnp.zeros((_B,), jnp.int32)
    rank0 = jnp.full((_B, 1, 1), float(_K), jnp.float32)

    h1 = _make_sc_hist(1)(xr, z16)
    p1, r1 = _loc1_call(h1.reshape(_B, 2, _NBS, 128), rank0)
    h2 = _make_sc_hist(2)(xr, p1.reshape(_B))
    return _loc2_mask_call(h2.reshape(_B, 2, _NBS, 128), r1, p1, x)


# speedup vs baseline: 13.8752x; 1.0395x over previous
import functools

import jax
import jax.numpy as jnp
from jax import lax
from jax.experimental import pallas as pl
from jax.experimental.pallas import tpu as pltpu
from jax.experimental.pallas import tpu_sc as plsc

_B = 16
_N = 96 * 56 * 56
_K = int(0.05 * _N)
_HALF = _N // 2
_CHUNK = _HALF // 8
_NCHUNK = _HALF // _CHUNK
_NBINS = 65536
_NBS = _NBINS // 128


def _make_sc_hist(pass_idx):
    mesh = plsc.VectorSubcoreMesh(core_axis_name="c", subcore_axis_name="s")

    @functools.partial(
        pl.kernel,
        mesh=mesh,
        compiler_params=pltpu.CompilerParams(needs_layout_passes=False),
        out_type=jax.ShapeDtypeStruct((_B, 2, _NBINS), jnp.int32),
        scratch_types=[
            pltpu.VMEM((_CHUNK,), jnp.float32),
            pltpu.VMEM((_CHUNK,), jnp.float32),
            pltpu.VMEM((_NBINS,), jnp.int32),
            pltpu.VMEM((16,), jnp.int32),
            pltpu.SemaphoreType.DMA,
            pltpu.SemaphoreType.DMA,
        ],
    )
    def hist_kernel(x_hbm, pref_hbm, out_hbm, buf0, buf1, hist, pvec, sem0,
                    sem1):
        row = lax.axis_index("s")
        half = lax.axis_index("c")
        base = half * _HALF

        bufs = (buf0, buf1)
        sems = (sem0, sem1)
        cur = pltpu.async_copy(x_hbm.at[row, pl.ds(base, _CHUNK)], buf0, sem0)

        if pass_idx > 1:
            pltpu.sync_copy(pref_hbm, pvec)
            rowv = jnp.zeros((16,), jnp.int32) + row
            pbroad = plsc.load_gather(pvec, [rowv])

        zer = jnp.zeros((16,), jnp.int32)

        def zbody(i, carry):
            hist[pl.ds(i * 16, 16)] = zer
            return carry

        lax.fori_loop(0, _NBINS // 16, zbody, 0, unroll=8)

        ones = jnp.ones((16,), jnp.int32)

        for j in range(_NCHUNK):
            nxt = None
            if j + 1 < _NCHUNK:
                nxt = pltpu.async_copy(
                    x_hbm.at[row, pl.ds(base + (j + 1) * _CHUNK, _CHUNK)],
                    bufs[(j + 1) % 2], sems[(j + 1) % 2])
            cur.wait()
            buf = bufs[j % 2]

            if pass_idx == 1:

                def body(i, carry):
                    v = buf[pl.ds(i * 16, 16)]
                    b = plsc.bitcast(v, jnp.int32)
                    key = b ^ (jnp.right_shift(b, 31) & jnp.int32(0x7FFFFFFF))
                    bin_ = jnp.right_shift(key, 16) + jnp.int32(32768)
                    plsc.addupdate_scatter(hist, [bin_], ones)
                    return carry

                lax.fori_loop(0, _CHUNK // 16, body, 0, unroll=8)
            else:

                def body(i, carry):
                    bins = []
                    masks = []
                    anym = None
                    for u in range(16):
                        v = buf[pl.ds(i * 256 + u * 16, 16)]
                        b = plsc.bitcast(v, jnp.int32)
                        key = b ^ (jnp.right_shift(b, 31)
                                   & jnp.int32(0x7FFFFFFF))
                        pm = jnp.right_shift(key, 16) == pbroad
                        bins.append(key & jnp.int32(0xFFFF))
                        masks.append(pm)
                        anym = pm if anym is None else (anym | pm)
                    cnt = jnp.sum(anym.astype(jnp.int32))

                    def do_scatter(_):
                        for u in range(16):
                            plsc.addupdate_scatter(hist, [bins[u]], ones,
                                                   mask=masks[u])
                        return 0

                    lax.cond(cnt > 0, do_scatter, lambda _: 0, 0)
                    return carry

                lax.fori_loop(0, _CHUNK // 256, body, 0, unroll=1)
            cur = nxt

        pltpu.sync_copy(hist, out_hbm.at[row, half])

    return hist_kernel


def _locate(hist_ref, r_in):
    c = (hist_ref[0, 0] + hist_ref[0, 1]).astype(jnp.float32)

    def _s11(a):
        return jnp.sum(jnp.sum(a, axis=1, keepdims=True), axis=0,
                       keepdims=True)

    rsum = jnp.sum(c, axis=1, keepdims=True)
    ts = rsum
    d = 1
    while d < _NBS:
        ts = ts + jnp.concatenate(
            [ts[d:, :], jnp.zeros((d, 1), jnp.float32)], axis=0)
        d *= 2
    i_star = jnp.sum(jnp.sum((ts >= r_in).astype(jnp.float32), axis=0,
                             keepdims=True), axis=1, keepdims=True) - 1.0
    ii = lax.broadcasted_iota(jnp.int32, (_NBS, 1), 0).astype(jnp.float32)
    ts_next = jnp.sum(jnp.where(ii == i_star + 1.0, ts, 0.0), axis=0,
                      keepdims=True)
    r2 = r_in - ts_next

    ii2 = lax.broadcasted_iota(jnp.int32, (_NBS, 128), 0).astype(jnp.float32)
    crow = jnp.sum(jnp.where(ii2 == i_star, c, 0.0), axis=0,
                   keepdims=True)
    ls = crow
    d = 1
    while d < 128:
        ls = ls + jnp.concatenate(
            [ls[:, d:], jnp.zeros((1, d), jnp.float32)], axis=1)
        d *= 2
    j_star = jnp.sum((ls >= r2).astype(jnp.float32), axis=1,
                     keepdims=True) - 1.0
    jj = lax.broadcasted_iota(jnp.int32, (1, 128), 1).astype(jnp.float32)
    ls_next = jnp.sum(jnp.where(jj == j_star + 1.0, ls, 0.0), axis=1,
                      keepdims=True)
    b_star = i_star * 128.0 + j_star
    above = ts_next + ls_next
    return b_star, above


def _loc1_call(hist4, rank):
    def body(hist_ref, rank_ref, p_out, r_out):
        r_in = rank_ref[0]
        b_star, above = _locate(hist_ref, r_in)
        p_out[0] = b_star.astype(jnp.int32) - jnp.int32(32768)
        r_out[0] = r_in - above

    return pl.pallas_call(
        body,
        grid=(_B,),
        in_specs=[
            pl.BlockSpec((1, 2, _NBS, 128), lambda r: (r, 0, 0, 0)),
            pl.BlockSpec((1, 1, 1), lambda r: (r, 0, 0)),
        ],
        out_specs=[
            pl.BlockSpec((1, 1, 1), lambda r: (r, 0, 0)),
            pl.BlockSpec((1, 1, 1), lambda r: (r, 0, 0)),
        ],
        out_shape=[
            jax.ShapeDtypeStruct((_B, 1, 1), jnp.int32),
            jax.ShapeDtypeStruct((_B, 1, 1), jnp.float32),
        ],
    )(hist4, rank)


def _loc2_mask_call(hist4, rank, pref, x):
    def body(hist_ref, rank_ref, pref_ref, x_ref, o_ref):
        r_in = rank_ref[0]
        b_star, _ = _locate(hist_ref, r_in)
        key = pref_ref[0] * jnp.int32(65536) + b_star.astype(jnp.int32)
        bits = key ^ (jnp.right_shift(key, 31) & jnp.int32(0x7FFFFFFF))
        thr = lax.bitcast_convert_type(bits, jnp.float32)
        t4 = jnp.reshape(thr, (1, 1, 1, 1))
        xv = x_ref[...]
        o_ref[...] = xv * (xv >= t4).astype(xv.dtype)

    return pl.pallas_call(
        body,
        grid=(_B,),
        in_specs=[
            pl.BlockSpec((1, 2, _NBS, 128), lambda r: (r, 0, 0, 0)),
            pl.BlockSpec((1, 1, 1), lambda r: (r, 0, 0)),
            pl.BlockSpec((1, 1, 1), lambda r: (r, 0, 0)),
            pl.BlockSpec((1, 96, 56, 56), lambda r: (r, 0, 0, 0)),
        ],
        out_specs=pl.BlockSpec((1, 96, 56, 56), lambda r: (r, 0, 0, 0)),
        out_shape=jax.ShapeDtypeStruct((_B, 96, 56, 56), jnp.float32),
    )(hist4, rank, pref, x)


def kernel(x):
    assert x.shape == (_B, 96, 56, 56) and x.dtype == jnp.float32
    xr = x.reshape(_B, _N)
    z16 = jnp.zeros((_B,), jnp.int32)
    rank0 = jnp.full((_B, 1, 1), float(_K), jnp.float32)

    h1 = _make_sc_hist(1)(xr, z16)
    p1, r1 = _loc1_call(h1.reshape(_B, 2, _NBS, 128), rank0)
    h2 = _make_sc_hist(2)(xr, p1.reshape(_B))
    return _loc2_mask_call(h2.reshape(_B, 2, _NBS, 128), r1, p1, x)
